# baseline (speedup 1.0000x reference)
import functools
import math

import numpy as np

import jax
import jax.numpy as jnp
from jax import lax
from jax.experimental import pallas as pl
from jax.experimental.pallas import tpu as pltpu
from jax.experimental.pallas import tpu_sc as plsc

VOCAB = 100000
D_MODEL = 1024
BATCH = 4
SEQ = 4096
SCALE = math.sqrt(D_MODEL)

NC = 2
NS = 16
NW = NC * NS
S_PER_W = SEQ // NW
CH = 8
NCHUNK = S_PER_W // CH
R = BATCH * CH
GRP = 3
NROUND = 5
LANES = 16
K = D_MODEL // LANES
NPE = 3


def _pe_matrix():
    pos = np.arange(SEQ, dtype=np.float64).reshape(-1, 1)
    emb = np.arange(D_MODEL, dtype=np.float64) * 2.0 / D_MODEL
    emb = np.power(10000.0, emb)
    xm = pos / emb
    pe = np.zeros((SEQ, D_MODEL), dtype=np.float64)
    pe[:, 0::2] = np.sin(xm[:, 0::2])
    pe[:, 1::2] = np.cos(xm[:, 1::2])
    return pe.astype(np.float32)


_PE = _pe_matrix()

_MESH = plsc.VectorSubcoreMesh(core_axis_name="c", subcore_axis_name="s")


@functools.partial(
    pl.kernel,
    out_type=jax.ShapeDtypeStruct((BATCH * SEQ, D_MODEL), jnp.float32),
    mesh=_MESH,
    scratch_types=[
        pltpu.VMEM((NCHUNK, R), jnp.int32),
        pltpu.VMEM((NPE, CH, D_MODEL), jnp.float32),
        pltpu.VMEM((GRP, BATCH, CH, D_MODEL), jnp.float32),
        pltpu.SemaphoreType.DMA,
        pltpu.SemaphoreType.DMA,
        pltpu.SemaphoreType.DMA,
    ],
)
def _sc_embed(x_hbm, table_hbm, pe_hbm, out_hbm, idx_v, pe_v, rows_v,
              gsem, psem, ssem):
    wid = lax.axis_index("s") * NC + lax.axis_index("c")
    s0 = wid * S_PER_W

    pltpu.sync_copy(x_hbm.at[wid], idx_v)

    def gather_copies(t, slot):
        return [
            pltpu.make_async_copy(
                table_hbm.at[idx_v.at[t, pl.ds(b * CH, CH)]],
                rows_v.at[slot, b],
                gsem,
            )
            for b in range(BATCH)
        ]

    def pe_copy(t, slot):
        return pltpu.make_async_copy(
            pe_hbm.at[pl.ds(s0 + t * CH, CH)], pe_v.at[slot], psem
        )

    def store_copies(t, slot):
        return [
            pltpu.make_async_copy(
                rows_v.at[slot, b],
                out_hbm.at[pl.ds(b * SEQ + s0 + t * CH, CH)],
                ssem,
            )
            for b in range(BATCH)
        ]

    def compute_chunk(slot, pslot):
        def vec_body(k, _):
            off = pl.ds(k * LANES, LANES)
            for i in range(CH):
                pvec = pe_v[pslot, i, off]
                for b in range(BATCH):
                    sl = (slot, b, i, off)
                    rows_v[sl] = rows_v[sl] * SCALE + pvec
            return 0

        lax.fori_loop(0, K, vec_body, 0)

    for cp in gather_copies(0, 0) + gather_copies(1, 1):
        cp.start()
    pe_copy(0, 0).start()
    pe_copy(1, 1).start()

    def round_body(r, _):
        for j in range(GRP):
            t = r * GRP + j

            for cp in gather_copies(t, j):
                cp.wait()
            pe_copy(t, j).wait()
            compute_chunk(j, j)
            for cp in store_copies(t, j):
                cp.start()

            if j > 0:
                for cp in store_copies(t - 1, j - 1):
                    cp.wait()
            else:
                @pl.when(r >= 1)
                def _(t=t):
                    for cp in store_copies(t - 1, GRP - 1):
                        cp.wait()

            def prefetch(t=t, j=j):
                for cp in gather_copies(t + 2, (j + 2) % GRP):
                    cp.start()
                pe_copy(t + 2, (j + 2) % NPE).start()

            if j < GRP - 1:
                prefetch()
            else:
                pl.when(r < NROUND - 1)(prefetch)

        return 0

    lax.fori_loop(0, NROUND, round_body, 0)

    t_tail = NROUND * GRP
    for cp in gather_copies(t_tail, 0):
        cp.wait()
    pe_copy(t_tail, 0).wait()
    compute_chunk(0, 0)
    for cp in store_copies(t_tail, 0):
        cp.start()
    for cp in store_copies(t_tail - 1, GRP - 1):
        cp.wait()
    for cp in store_copies(t_tail, 0):
        cp.wait()


def kernel(x, table):
    x_r = (
        x.reshape(BATCH, NW, NCHUNK, CH)
        .transpose(1, 2, 0, 3)
        .reshape(NW, NCHUNK, R)
    )
    out = _sc_embed(x_r, table, jnp.asarray(_PE))
    return out.reshape(BATCH, SEQ, D_MODEL)


# speedup vs baseline: 1.0183x; 1.0183x over previous
import functools
import math

import numpy as np

import jax
import jax.numpy as jnp
from jax import lax
from jax.experimental import pallas as pl
from jax.experimental.pallas import tpu as pltpu
from jax.experimental.pallas import tpu_sc as plsc

VOCAB = 100000
D_MODEL = 1024
BATCH = 4
SEQ = 4096
SCALE = math.sqrt(D_MODEL)

NC = 2
NS = 16
NW = NC * NS
S_PER_W = SEQ // NW
CH = 4
NCHUNK = S_PER_W // CH
R = BATCH * CH
GRP = 5
NROUND = 6
LANES = 16
K = D_MODEL // LANES
NPE = 5


def _pe_matrix():
    pos = np.arange(SEQ, dtype=np.float64).reshape(-1, 1)
    emb = np.arange(D_MODEL, dtype=np.float64) * 2.0 / D_MODEL
    emb = np.power(10000.0, emb)
    xm = pos / emb
    pe = np.zeros((SEQ, D_MODEL), dtype=np.float64)
    pe[:, 0::2] = np.sin(xm[:, 0::2])
    pe[:, 1::2] = np.cos(xm[:, 1::2])
    return pe.astype(np.float32)


_PE = _pe_matrix()

_MESH = plsc.VectorSubcoreMesh(core_axis_name="c", subcore_axis_name="s")


@functools.partial(
    pl.kernel,
    out_type=jax.ShapeDtypeStruct((BATCH * SEQ, D_MODEL), jnp.float32),
    mesh=_MESH,
    scratch_types=[
        pltpu.VMEM((NCHUNK, R), jnp.int32),
        pltpu.VMEM((NPE, CH, D_MODEL), jnp.float32),
        pltpu.VMEM((GRP, BATCH, CH, D_MODEL), jnp.float32),
        pltpu.SemaphoreType.DMA,
        pltpu.SemaphoreType.DMA,
        pltpu.SemaphoreType.DMA,
    ],
)
def _sc_embed(x_hbm, table_hbm, pe_hbm, out_hbm, idx_v, pe_v, rows_v,
              gsem, psem, ssem):
    wid = lax.axis_index("s") * NC + lax.axis_index("c")
    s0 = wid * S_PER_W

    pltpu.sync_copy(x_hbm.at[wid], idx_v)

    def gather_copies(t, slot):
        return [
            pltpu.make_async_copy(
                table_hbm.at[idx_v.at[t, pl.ds(b * CH, CH)]],
                rows_v.at[slot, b],
                gsem,
            )
            for b in range(BATCH)
        ]

    def pe_copy(t, slot):
        return pltpu.make_async_copy(
            pe_hbm.at[pl.ds(s0 + t * CH, CH)], pe_v.at[slot], psem
        )

    def store_copies(t, slot):
        return [
            pltpu.make_async_copy(
                rows_v.at[slot, b],
                out_hbm.at[pl.ds(b * SEQ + s0 + t * CH, CH)],
                ssem,
            )
            for b in range(BATCH)
        ]

    def compute_chunk(slot, pslot):
        def vec_body(k, _):
            off = pl.ds(k * LANES, LANES)
            for i in range(CH):
                pvec = pe_v[pslot, i, off]
                for b in range(BATCH):
                    sl = (slot, b, i, off)
                    rows_v[sl] = rows_v[sl] * SCALE + pvec
            return 0

        lax.fori_loop(0, K, vec_body, 0)

    for tt in range(4):
        for cp in gather_copies(tt, tt):
            cp.start()
    pe_copy(0, 0).start()
    pe_copy(1, 1).start()

    def round_body(r, _):
        for j in range(GRP):
            t = r * GRP + j

            for cp in gather_copies(t, j):
                cp.wait()
            pe_copy(t, j).wait()
            compute_chunk(j, j)
            for cp in store_copies(t, j):
                cp.start()

            if j > 0:
                for cp in store_copies(t - 1, j - 1):
                    cp.wait()
            else:
                @pl.when(r >= 1)
                def _(t=t):
                    for cp in store_copies(t - 1, GRP - 1):
                        cp.wait()

            def pre_gather(t=t, j=j):
                for cp in gather_copies(t + 4, (j + 4) % GRP):
                    cp.start()

            if j < 3:
                pre_gather()
            else:
                pl.when(r < NROUND - 1)(pre_gather)
            pe_copy(t + 2, (j + 2) % NPE).start()

        return 0

    lax.fori_loop(0, NROUND, round_body, 0)

    t30 = NROUND * GRP
    for cp in gather_copies(t30, 0):
        cp.wait()
    pe_copy(t30, 0).wait()
    compute_chunk(0, 0)
    for cp in store_copies(t30, 0):
        cp.start()
    for cp in store_copies(t30 - 1, GRP - 1):
        cp.wait()
    t31 = t30 + 1
    for cp in gather_copies(t31, 1):
        cp.wait()
    pe_copy(t31, 1).wait()
    compute_chunk(1, 1)
    for cp in store_copies(t31, 1):
        cp.start()
    for cp in store_copies(t30, 0):
        cp.wait()
    for cp in store_copies(t31, 1):
        cp.wait()


def kernel(x, table):
    x_r = (
        x.reshape(BATCH, NW, NCHUNK, CH)
        .transpose(1, 2, 0, 3)
        .reshape(NW, NCHUNK, R)
    )
    out = _sc_embed(x_r, table, jnp.asarray(_PE))
    return out.reshape(BATCH, SEQ, D_MODEL)


# speedup vs baseline: 1.0236x; 1.0052x over previous
import functools
import math

import numpy as np

import jax
import jax.numpy as jnp
from jax import lax
from jax.experimental import pallas as pl
from jax.experimental.pallas import tpu as pltpu
from jax.experimental.pallas import tpu_sc as plsc

VOCAB = 100000
D_MODEL = 1024
BATCH = 4
SEQ = 4096
SCALE = math.sqrt(D_MODEL)

NC = 2
NS = 16
NW = NC * NS
S_PER_W = SEQ // NW
CH = 4
NCHUNK = S_PER_W // CH
R = BATCH * CH
GRP = 4
NROUND = NCHUNK // GRP
LANES = 16
K = D_MODEL // LANES
NPE = 2


def _pe_matrix():
    pos = np.arange(SEQ, dtype=np.float64).reshape(-1, 1)
    emb = np.arange(D_MODEL, dtype=np.float64) * 2.0 / D_MODEL
    emb = np.power(10000.0, emb)
    xm = pos / emb
    pe = np.zeros((SEQ, D_MODEL), dtype=np.float64)
    pe[:, 0::2] = np.sin(xm[:, 0::2])
    pe[:, 1::2] = np.cos(xm[:, 1::2])
    return pe.astype(np.float32)


_PE = _pe_matrix()

_MESH = plsc.VectorSubcoreMesh(core_axis_name="c", subcore_axis_name="s")


@functools.partial(
    pl.kernel,
    out_type=jax.ShapeDtypeStruct((BATCH * SEQ, D_MODEL), jnp.float32),
    mesh=_MESH,
    scratch_types=[
        pltpu.VMEM((NCHUNK, R), jnp.int32),
        pltpu.VMEM((NPE, CH, D_MODEL), jnp.float32),
        pltpu.VMEM((GRP, BATCH, CH, D_MODEL), jnp.float32),
        pltpu.SemaphoreType.DMA,
        pltpu.SemaphoreType.DMA,
        pltpu.SemaphoreType.DMA,
    ],
)
def _sc_embed(x_hbm, table_hbm, pe_hbm, out_hbm, idx_v, pe_v, rows_v,
              gsem, psem, ssem):
    wid = lax.axis_index("s") * NC + lax.axis_index("c")
    s0 = wid * S_PER_W

    pltpu.sync_copy(x_hbm.at[wid], idx_v)

    def gather_copies(t, slot):
        return [
            pltpu.make_async_copy(
                table_hbm.at[idx_v.at[t, pl.ds(b * CH, CH)]],
                rows_v.at[slot, b],
                gsem,
            )
            for b in range(BATCH)
        ]

    def pe_copy(t, slot):
        return pltpu.make_async_copy(
            pe_hbm.at[pl.ds(s0 + t * CH, CH)], pe_v.at[slot], psem
        )

    def store_copies(t, slot):
        return [
            pltpu.make_async_copy(
                rows_v.at[slot, b],
                out_hbm.at[pl.ds(b * SEQ + s0 + t * CH, CH)],
                ssem,
            )
            for b in range(BATCH)
        ]

    for cp in gather_copies(0, 0) + gather_copies(1, 1) + gather_copies(2, 2):
        cp.start()
    pe_copy(0, 0).start()
    pe_copy(1, 1).start()

    def round_body(r, _):
        for j in range(GRP):
            t = r * GRP + j
            pslot = j % NPE

            for cp in gather_copies(t, j):
                cp.wait()
            pe_copy(t, pslot).wait()

            def vec_body(k, _, j=j, pslot=pslot):
                off = pl.ds(k * LANES, LANES)
                for i in range(CH):
                    pvec = pe_v[pslot, i, off]
                    for b in range(BATCH):
                        sl = (j, b, i, off)
                        rows_v[sl] = rows_v[sl] * SCALE + pvec
                return 0

            lax.fori_loop(0, K, vec_body, 0)

            for cp in store_copies(t, j):
                cp.start()

            if j > 0:
                for cp in store_copies(t - 1, j - 1):
                    cp.wait()
            else:
                @pl.when(r >= 1)
                def _(t=t):
                    for cp in store_copies(t - 1, GRP - 1):
                        cp.wait()

            def pre_gather(t=t, j=j):
                for cp in gather_copies(t + 3, (j + 3) % GRP):
                    cp.start()

            def pre_pe(t=t, pslot=pslot):
                pe_copy(t + 2, pslot).start()

            if j == 0:
                pre_gather()
            else:
                pl.when(r < NROUND - 1)(pre_gather)
            if j < 2:
                pre_pe()
            else:
                pl.when(r < NROUND - 1)(pre_pe)

        return 0

    lax.fori_loop(0, NROUND, round_body, 0)

    for cp in store_copies(NCHUNK - 1, GRP - 1):
        cp.wait()


def kernel(x, table):
    x_r = (
        x.reshape(BATCH, NW, NCHUNK, CH)
        .transpose(1, 2, 0, 3)
        .reshape(NW, NCHUNK, R)
    )
    out = _sc_embed(x_r, table, jnp.asarray(_PE))
    return out.reshape(BATCH, SEQ, D_MODEL)
